# Initial kernel scaffold; baseline (speedup 1.0000x reference)
import jax
import jax.numpy as jnp
from jax.experimental import pallas as pl


def kernel(x, edge_index, edge_attr, params):
    raise NotImplementedError("write your pallas kernel here")


# speedup vs baseline: 34.9984x; 34.9984x over previous
import dataclasses
import functools

import jax
import jax.numpy as jnp
from jax import lax
from jax.experimental import pallas as pl
from jax.experimental.pallas import tpu as pltpu
from jax.experimental.pallas import tpu_sc as plsc

N = 10000
E = 320000
IN_DIM = 128
HID = 16
HEADS = 8
NUM_LAYERS = 4
EDGE_DIM = 16
HD = HID * HEADS

NP = 10112
TRASH = N
K = 128
NCORE = 2
NSUB = 16
NTILE = NCORE * NSUB
ETOT = E + N
CHUNKS = 2592
CPT = CHUNKS // NTILE
SELF_CHUNK0 = E // K
ROWS_PT = NP // NSUB

_HIGH = lax.Precision.HIGHEST


def _dot(a, b):
    return jnp.dot(a, b, preferred_element_type=jnp.float32, precision=_HIGH)


def _k_in_body(x_ref, w_ref, b_ref, g_ref, bb_ref, o_ref):
    h = _dot(x_ref[...], w_ref[...]) + b_ref[...]
    m = jnp.mean(h, axis=-1, keepdims=True)
    v = jnp.mean((h - m) ** 2, axis=-1, keepdims=True)
    h = (h - m) / jnp.sqrt(v + 1e-5) * g_ref[...] + bb_ref[...]
    o_ref[...] = jnp.maximum(h, 0.0)


def _k_pre_body(h_ref, w_ref, as_ref, ad_ref, hw_ref, s16_ref, d16_ref):
    hw = _dot(h_ref[...], w_ref[...])
    hw_ref[...] = hw
    s16_ref[...] = _dot(hw, as_ref[...])
    d16_ref[...] = _dot(hw, ad_ref[...])


def _k_edge_body(ea_ref, we_ref, p_ref):
    p_ref[...] = _dot(ea_ref[...], we_ref[...])


def _k_mean_body(ea_ref, o_ref):
    @pl.when(pl.program_id(0) == 0)
    def _():
        o_ref[...] = jnp.zeros_like(o_ref)

    s = jnp.sum(ea_ref[...], axis=0, keepdims=True)
    o_ref[...] += jnp.broadcast_to(s, o_ref.shape)


def _k_pad_body(ms_ref, we_ref, o_ref):
    m = ms_ref[0:1, :] * (1.0 / E)
    for l in range(NUM_LAYERS):
        v = _dot(m, we_ref[l])
        o_ref[l] = jnp.broadcast_to(v, (K, EDGE_DIM))


def _k_post_body(acc_ref, den_ref, r_ref, b_ref, g_ref, bb_ref, res_ref,
                 o_ref, *, use_res):
    a = acc_ref[0] + acc_ref[1]
    den = den_ref[0] + den_ref[1] + 1e-16
    dene = _dot(den, r_ref[...])
    o = a / dene + b_ref[...]
    m = jnp.mean(o, axis=-1, keepdims=True)
    v = jnp.mean((o - m) ** 2, axis=-1, keepdims=True)
    o = (o - m) / jnp.sqrt(v + 1e-5) * g_ref[...] + bb_ref[...]
    if use_res:
        o = o + res_ref[...]
    o_ref[...] = jnp.where(o > 0, o, jnp.exp(o) - 1.0)


def _k_final_body(acc_ref, den_ref, r_ref, m_ref, b_ref, g_ref, bb_ref, o_ref):
    a = acc_ref[0] + acc_ref[1]
    den = den_ref[0] + den_ref[1] + 1e-16
    dene = _dot(den, r_ref[...])
    o = _dot(a / dene, m_ref[...]) + b_ref[...]
    m = jnp.mean(o, axis=-1, keepdims=True)
    v = jnp.mean((o - m) ** 2, axis=-1, keepdims=True)
    o_ref[...] = (o - m) / jnp.sqrt(v + 1e-5) * g_ref[...] + bb_ref[...]


_ROWS_B = 1000
_GRID_N = N // _ROWS_B
_EB = 4000
_GRID_E = E // _EB


def _full(shape):
    return pl.BlockSpec(shape, lambda i: (0,) * len(shape))


def _rows(shape):
    return pl.BlockSpec(shape, lambda i: (i,) + (0,) * (len(shape) - 1))


def _sc_layer(hw, a_src16, a_dst16, edge_index, p_l, aepad_l):
    mesh = plsc.VectorSubcoreMesh(
        core_axis_name="c", subcore_axis_name="s",
        num_cores=NCORE, num_subcores=NSUB)

    cp = pltpu.CompilerParams()
    if "needs_layout_passes" in pltpu.CompilerParams.__dataclass_fields__:
        cp = dataclasses.replace(cp, needs_layout_passes=False)
    if "use_tc_tiling_on_sc" in pltpu.CompilerParams.__dataclass_fields__:
        cp = dataclasses.replace(cp, use_tc_tiling_on_sc=False)

    @functools.partial(
        pl.kernel,
        out_type=[
            jax.ShapeDtypeStruct((NCORE, NP, HD), jnp.float32),
            jax.ShapeDtypeStruct((NCORE, NP, EDGE_DIM), jnp.float32),
        ],
        mesh=mesh,
        scratch_types=[
            pltpu.VMEM_SHARED((NP, HD), jnp.float32),
            pltpu.VMEM_SHARED((NP, EDGE_DIM), jnp.float32),
            pltpu.VMEM((K, HD), jnp.float32),
            pltpu.VMEM((K, EDGE_DIM), jnp.float32),
            pltpu.VMEM((K, EDGE_DIM), jnp.float32),
            pltpu.VMEM((K, EDGE_DIM), jnp.float32),
            pltpu.VMEM((K, EDGE_DIM), jnp.float32),
            pltpu.VMEM((K,), jnp.int32),
            pltpu.VMEM((K,), jnp.int32),
        ],
        compiler_params=cp,
    )
    def k(hw_hbm, as_hbm, ad_hbm, ei_hbm, p_hbm, aep_hbm, acc_out, den_out,
          acc_sp, den_sp, hb, ab, bb, aeb, exb, sb, db):
        c = lax.axis_index("c")
        s = lax.axis_index("s")
        w = c * NSUB + s

        @pl.loop(0, K)
        def _(i):
            for j in range(HD // 16):
                hb[i, pl.ds(j * 16, 16)] = jnp.zeros((16,), jnp.float32)
            exb[i, :] = jnp.zeros((16,), jnp.float32)

        r0 = s * ROWS_PT
        nfull = ROWS_PT // K
        rem = ROWS_PT - nfull * K
        for q in range(nfull):
            pltpu.sync_copy(hb, acc_sp.at[pl.ds(r0 + q * K, K)])
            pltpu.sync_copy(exb, den_sp.at[pl.ds(r0 + q * K, K)])
        pltpu.sync_copy(hb.at[pl.ds(0, rem)],
                        acc_sp.at[pl.ds(r0 + nfull * K, rem)])
        pltpu.sync_copy(exb.at[pl.ds(0, rem)],
                        den_sp.at[pl.ds(r0 + nfull * K, rem)])
        plsc.subcore_barrier()

        @pl.loop(0, CPT)
        def _(tt):
            t = w * CPT + tt
            base = t * K

            @pl.when(t < SELF_CHUNK0)
            def _():
                pltpu.sync_copy(ei_hbm.at[0, pl.ds(base, K)], sb)
                pltpu.sync_copy(ei_hbm.at[1, pl.ds(base, K)], db)
                pltpu.sync_copy(p_hbm.at[pl.ds(base, K)], aeb)

            @pl.when(t >= SELF_CHUNK0)
            def _():
                @pl.loop(0, K // 16)
                def _(j):
                    v = (base - E + j * 16) + lax.iota(jnp.int32, 16)
                    sb[pl.ds(j * 16, 16)] = jnp.minimum(v, N - 1)
                    db[pl.ds(j * 16, 16)] = jnp.minimum(v, TRASH)
                pltpu.sync_copy(aep_hbm, aeb)

            pltpu.sync_copy(as_hbm.at[sb], ab)
            pltpu.sync_copy(ad_hbm.at[db], bb)
            pltpu.sync_copy(hw_hbm.at[sb], hb)

            lane = lax.iota(jnp.int32, 16)

            @pl.loop(0, K)
            def _(i):
                al = ab[i, :] + bb[i, :] + aeb[i, :]
                al = jnp.where(al >= 0.0, al, al * 0.2)
                ex = jnp.exp(al)
                exb[i, :] = ex
                for j in range(HEADS):
                    sj = jnp.sum(jnp.where(lane == j, ex, 0.0))
                    sl = pl.ds(j * 16, 16)
                    hb[i, sl] = hb[i, sl] * sj

            pltpu.sync_copy(hb, acc_sp.at[db], add=True)
            pltpu.sync_copy(exb, den_sp.at[db], add=True)

        plsc.subcore_barrier()
        pltpu.sync_copy(acc_sp.at[pl.ds(r0, ROWS_PT)],
                        acc_out.at[c, pl.ds(r0, ROWS_PT)])
        pltpu.sync_copy(den_sp.at[pl.ds(r0, ROWS_PT)],
                        den_out.at[c, pl.ds(r0, ROWS_PT)])

    return k(hw, a_src16, a_dst16, edge_index, p_l, aepad_l)


def _att_fold(att):
    flat = att[0].reshape(HD)
    h_of = jnp.arange(HD, dtype=jnp.int32) // HID
    return flat[:, None] * jax.nn.one_hot(h_of, EDGE_DIM, dtype=jnp.float32)


def kernel(x, edge_index, edge_attr, params):
    layers = params["layers"]

    we_all = jnp.stack([
        jnp.pad(
            jnp.sum(p["W_edge"].reshape(EDGE_DIM, HEADS, HID)
                    * p["att_edge"][0][None], axis=-1),
            ((0, 0), (0, EDGE_DIM - HEADS)))
        for p in layers])
    a_src_m = [_att_fold(p["att_src"]) for p in layers]
    a_dst_m = [_att_fold(p["att_dst"]) for p in layers]

    h_of = jnp.arange(HD, dtype=jnp.int32) // HID
    d_of = jnp.arange(HD, dtype=jnp.int32) % HID
    rmat = jax.nn.one_hot(h_of, EDGE_DIM, dtype=jnp.float32).T
    mmat = jax.nn.one_hot(d_of, HID, dtype=jnp.float32) / HEADS

    r2 = lambda v: v.reshape(1, -1)

    h = pl.pallas_call(
        _k_in_body,
        grid=(_GRID_N,),
        in_specs=[_rows((_ROWS_B, IN_DIM)), _full((IN_DIM, HD)),
                  _full((1, HD)), _full((1, HD)), _full((1, HD))],
        out_specs=_rows((_ROWS_B, HD)),
        out_shape=jax.ShapeDtypeStruct((N, HD), jnp.float32),
    )(x, params["W_in"], r2(params["b_in"]),
      r2(params["ln_in_g"]), r2(params["ln_in_b"]))

    p_all = [
        pl.pallas_call(
            _k_edge_body,
            grid=(_GRID_E,),
            in_specs=[_rows((_EB, EDGE_DIM)), _full((EDGE_DIM, EDGE_DIM))],
            out_specs=_rows((_EB, EDGE_DIM)),
            out_shape=jax.ShapeDtypeStruct((E, EDGE_DIM), jnp.float32),
        )(edge_attr, we_all[l])
        for l in range(NUM_LAYERS)
    ]

    msum = pl.pallas_call(
        _k_mean_body,
        grid=(_GRID_E,),
        in_specs=[_rows((_EB, EDGE_DIM))],
        out_specs=_full((8, EDGE_DIM)),
        out_shape=jax.ShapeDtypeStruct((8, EDGE_DIM), jnp.float32),
    )(edge_attr)

    aepad = pl.pallas_call(
        _k_pad_body,
        in_specs=[pl.BlockSpec((8, EDGE_DIM), lambda: (0, 0)),
                  pl.BlockSpec((NUM_LAYERS, EDGE_DIM, EDGE_DIM),
                               lambda: (0, 0, 0))],
        out_specs=pl.BlockSpec((NUM_LAYERS, K, EDGE_DIM), lambda: (0, 0, 0)),
        out_shape=jax.ShapeDtypeStruct((NUM_LAYERS, K, EDGE_DIM), jnp.float32),
    )(msum, we_all)

    for l in range(NUM_LAYERS):
        p = layers[l]
        hw, s16, d16 = pl.pallas_call(
            _k_pre_body,
            grid=(_GRID_N,),
            in_specs=[_rows((_ROWS_B, HD)), _full((HD, HD)),
                      _full((HD, EDGE_DIM)), _full((HD, EDGE_DIM))],
            out_specs=[_rows((_ROWS_B, HD)), _rows((_ROWS_B, EDGE_DIM)),
                       _rows((_ROWS_B, EDGE_DIM))],
            out_shape=[jax.ShapeDtypeStruct((N, HD), jnp.float32),
                       jax.ShapeDtypeStruct((N, EDGE_DIM), jnp.float32),
                       jax.ShapeDtypeStruct((N, EDGE_DIM), jnp.float32)],
        )(h, p["W"], a_src_m[l], a_dst_m[l])

        d16p = jnp.concatenate(
            [d16, jnp.zeros((NP - N, EDGE_DIM), jnp.float32)], axis=0)

        acc, den = _sc_layer(hw, s16, d16p, edge_index, p_all[l], aepad[l])

        if l < NUM_LAYERS - 1:
            h = pl.pallas_call(
                functools.partial(_k_post_body, use_res=(l > 0)),
                grid=(_GRID_N,),
                in_specs=[
                    pl.BlockSpec((NCORE, _ROWS_B, HD), lambda i: (0, i, 0)),
                    pl.BlockSpec((NCORE, _ROWS_B, EDGE_DIM),
                                 lambda i: (0, i, 0)),
                    _full((EDGE_DIM, HD)), _full((1, HD)), _full((1, HD)),
                    _full((1, HD)), _rows((_ROWS_B, HD))],
                out_specs=_rows((_ROWS_B, HD)),
                out_shape=jax.ShapeDtypeStruct((N, HD), jnp.float32),
            )(acc, den, rmat, r2(p["bias"]), r2(p["ln_g"]), r2(p["ln_b"]), h)
        else:
            h = pl.pallas_call(
                _k_final_body,
                grid=(_GRID_N,),
                in_specs=[
                    pl.BlockSpec((NCORE, _ROWS_B, HD), lambda i: (0, i, 0)),
                    pl.BlockSpec((NCORE, _ROWS_B, EDGE_DIM),
                                 lambda i: (0, i, 0)),
                    _full((EDGE_DIM, HD)), _full((HD, HID)),
                    _full((1, HID)), _full((1, HID)), _full((1, HID))],
                out_specs=_rows((_ROWS_B, HID)),
                out_shape=jax.ShapeDtypeStruct((N, HID), jnp.float32),
            )(acc, den, rmat, mmat, r2(p["bias"]), r2(p["ln_g"]),
              r2(p["ln_b"]))

    return h
